# baseline (speedup 1.0000x reference)
import functools

import jax
import jax.numpy as jnp
from jax.experimental import pallas as pl
from jax.experimental.pallas import tpu as pltpu

_EPS = 1e-5


def _cgb_kernel(x_hbm, wt_ref, gamma_ref, beta_ref, out_hbm,
                hbuf, sum8, sq8, sem_in, sem_out, *, n_rows, tile, num_tiles):
    ph = pl.program_id(0)
    t = pl.program_id(1)

    half = tile // 2

    def in_half(i, k):
        base = i * tile + k * half
        return pltpu.make_async_copy(
            x_hbm.at[pl.ds(base, half), :],
            hbuf.at[pl.ds(base, half), :],
            sem_in.at[jax.lax.rem(i, 2), k])

    def out_half(i, k):
        base = i * tile + k * half
        return pltpu.make_async_copy(
            hbuf.at[pl.ds(base, half), :],
            out_hbm.at[pl.ds(base, half), :],
            sem_out.at[jax.lax.rem(i, 2), k])

    def in_copy(i):
        class _Pair:
            def start(self):
                in_half(i, 0).start()
                in_half(i, 1).start()

            def wait(self):
                in_half(i, 0).wait()
                in_half(i, 1).wait()
        return _Pair()

    def out_copy(i):
        class _Pair:
            def start(self):
                out_half(i, 0).start()
                out_half(i, 1).start()

            def wait(self):
                out_half(i, 0).wait()
                out_half(i, 1).wait()
        return _Pair()

    @pl.when(ph == 0)
    def _stats_phase():
        @pl.when(t == 0)
        def _init():
            sum8[...] = jnp.zeros_like(sum8)
            sq8[...] = jnp.zeros_like(sq8)
            in_copy(0).start()
            in_copy(1).start()

        @pl.when((t >= 1) & (t <= num_tiles - 2))
        def _prefetch():
            in_copy(t + 1).start()

        in_copy(t).wait()
        xv = hbuf[pl.ds(t * tile, tile), :]
        h = jnp.dot(xv, wt_ref[...], preferred_element_type=jnp.float32)
        h3 = h.reshape(tile // 8, 8, 128)
        sum8[...] += jnp.sum(h3, axis=0)
        sq8[...] += jnp.sum(h3 * h3, axis=0)
        hbuf[pl.ds(t * tile, tile), :] = h

    @pl.when(ph == 1)
    def _apply_phase():
        inv_n = jnp.float32(1.0 / n_rows)
        mean = jnp.sum(sum8[...], axis=0, keepdims=True) * inv_n
        sq = jnp.sum(sq8[...], axis=0, keepdims=True) * inv_n
        var = sq - mean * mean
        scale = gamma_ref[...] * jax.lax.rsqrt(var + _EPS)
        shift = beta_ref[...] - mean * scale
        h = hbuf[pl.ds(t * tile, tile), :]
        hbuf[pl.ds(t * tile, tile), :] = jnp.maximum(h * scale + shift, 0.0)

        @pl.when(t > 0)
        def _drain_prev():
            out_copy(t - 1).wait()

        out_copy(t).start()

        @pl.when(t == num_tiles - 1)
        def _drain_last():
            out_copy(t).wait()


@jax.jit
def kernel(p, x, o, W, gamma, beta):
    del p, o
    n, din = x.shape
    dout = W.shape[0]
    tile = 10000
    assert n % tile == 0
    num_tiles = n // tile

    wt = W.T
    gamma2 = gamma.reshape(1, dout)
    beta2 = beta.reshape(1, dout)

    out = pl.pallas_call(
        functools.partial(_cgb_kernel, n_rows=n, tile=tile,
                          num_tiles=num_tiles),
        grid=(2, num_tiles),
        in_specs=[
            pl.BlockSpec(memory_space=pltpu.MemorySpace.HBM),
            pl.BlockSpec((din, dout), lambda ph, t: (0, 0)),
            pl.BlockSpec((1, dout), lambda ph, t: (0, 0)),
            pl.BlockSpec((1, dout), lambda ph, t: (0, 0)),
        ],
        out_specs=pl.BlockSpec(memory_space=pltpu.MemorySpace.HBM),
        out_shape=jax.ShapeDtypeStruct((n, dout), jnp.float32),
        scratch_shapes=[
            pltpu.VMEM((n, dout), jnp.float32),
            pltpu.VMEM((8, dout), jnp.float32),
            pltpu.VMEM((8, dout), jnp.float32),
            pltpu.SemaphoreType.DMA((2, 2)),
            pltpu.SemaphoreType.DMA((2, 2)),
        ],
        compiler_params=pltpu.CompilerParams(
            dimension_semantics=("arbitrary", "arbitrary"),
            vmem_limit_bytes=60 * 1024 * 1024,
        ),
    )(x, wt, gamma2, beta2)
    return out


# speedup vs baseline: 1.1146x; 1.1146x over previous
import functools

import jax
import jax.numpy as jnp
from jax.experimental import pallas as pl
from jax.experimental.pallas import tpu as pltpu

_EPS = 1e-5


def _cgb_kernel(x_hbm, wt_ref, gamma_ref, beta_ref, out_hbm,
                hbuf, sum8, sq8, sem_in, sem_out, *, n_rows, tile, num_tiles):
    ph = pl.program_id(0)
    t = pl.program_id(1)

    def in_copy(i):
        return pltpu.make_async_copy(
            x_hbm.at[pl.ds(i * tile, tile), :],
            hbuf.at[pl.ds(i * tile, tile), :],
            sem_in.at[jax.lax.rem(i, 2)])

    def out_copy(i):
        return pltpu.make_async_copy(
            hbuf.at[pl.ds(i * tile, tile), :],
            out_hbm.at[pl.ds(i * tile, tile), :],
            sem_out.at[jax.lax.rem(i, 2)])

    @pl.when(ph == 0)
    def _stats_phase():
        @pl.when(t == 0)
        def _init():
            sum8[...] = jnp.zeros_like(sum8)
            sq8[...] = jnp.zeros_like(sq8)
            in_copy(0).start()
            in_copy(1).start()

        @pl.when((t >= 1) & (t <= num_tiles - 2))
        def _prefetch():
            in_copy(t + 1).start()

        in_copy(t).wait()
        xv = hbuf[pl.ds(t * tile, tile), :]
        h = jnp.dot(xv, wt_ref[...], preferred_element_type=jnp.float32)
        h3 = h.reshape(tile // 8, 8, 128)
        sum8[...] += jnp.sum(h3, axis=0)
        sq8[...] += jnp.sum(h3 * h3, axis=0)
        hbuf[pl.ds(t * tile, tile), :] = h

    @pl.when(ph == 1)
    def _apply_phase():
        inv_n = jnp.float32(1.0 / n_rows)
        mean = jnp.sum(sum8[...], axis=0, keepdims=True) * inv_n
        sq = jnp.sum(sq8[...], axis=0, keepdims=True) * inv_n
        var = sq - mean * mean
        scale = gamma_ref[...] * jax.lax.rsqrt(var + _EPS)
        shift = beta_ref[...] - mean * scale
        h = hbuf[pl.ds(t * tile, tile), :]
        hbuf[pl.ds(t * tile, tile), :] = jnp.maximum(h * scale + shift, 0.0)

        @pl.when(t > 0)
        def _drain_prev():
            out_copy(t - 1).wait()

        out_copy(t).start()

        @pl.when(t == num_tiles - 1)
        def _drain_last():
            out_copy(t).wait()


@jax.jit
def kernel(p, x, o, W, gamma, beta):
    del p, o
    n, din = x.shape
    dout = W.shape[0]
    tile = 20000
    assert n % tile == 0
    num_tiles = n // tile

    wt = W.T
    gamma2 = gamma.reshape(1, dout)
    beta2 = beta.reshape(1, dout)

    out = pl.pallas_call(
        functools.partial(_cgb_kernel, n_rows=n, tile=tile,
                          num_tiles=num_tiles),
        grid=(2, num_tiles),
        in_specs=[
            pl.BlockSpec(memory_space=pltpu.MemorySpace.HBM),
            pl.BlockSpec((din, dout), lambda ph, t: (0, 0)),
            pl.BlockSpec((1, dout), lambda ph, t: (0, 0)),
            pl.BlockSpec((1, dout), lambda ph, t: (0, 0)),
        ],
        out_specs=pl.BlockSpec(memory_space=pltpu.MemorySpace.HBM),
        out_shape=jax.ShapeDtypeStruct((n, dout), jnp.float32),
        scratch_shapes=[
            pltpu.VMEM((n, dout), jnp.float32),
            pltpu.VMEM((8, dout), jnp.float32),
            pltpu.VMEM((8, dout), jnp.float32),
            pltpu.SemaphoreType.DMA((2,)),
            pltpu.SemaphoreType.DMA((2,)),
        ],
        compiler_params=pltpu.CompilerParams(
            dimension_semantics=("arbitrary", "arbitrary"),
            vmem_limit_bytes=60 * 1024 * 1024,
        ),
    )(x, wt, gamma2, beta2)
    return out
